# Initial kernel scaffold; baseline (speedup 1.0000x reference)
import jax
import jax.numpy as jnp
from jax.experimental import pallas as pl


def kernel(x, edge_index, W_emb, b_emb, ln1_g, ln1_b, ln2_g, ln2_b, Wq, bq, Wk, bk, Wv, bv, Wo, bo, W1, b1, W2, b2):
    raise NotImplementedError("write your pallas kernel here")


# speedup vs baseline: 1.2479x; 1.2479x over previous
import functools
import math

import jax
import jax.numpy as jnp
from jax import lax
from jax.experimental import pallas as pl
from jax.experimental.pallas import tpu as pltpu

N = 10000
E = 160000
D_IN = 256
D = 512
L = 4
C = 40

NPAD = 10240
ROWS_BLK = 1280
GRID_ROWS = NPAD // ROWS_BLK


def _embed_body(x_ref, w_ref, b_ref, o_ref):
    o_ref[...] = (
        jnp.dot(x_ref[...], w_ref[...], preferred_element_type=jnp.float32)
        + b_ref[...]
    )


def _embed(x_pad, W_emb, b_emb):
    return pl.pallas_call(
        _embed_body,
        grid=(GRID_ROWS,),
        in_specs=[
            pl.BlockSpec((ROWS_BLK, D_IN), lambda i: (i, 0)),
            pl.BlockSpec((D_IN, D), lambda i: (0, 0)),
            pl.BlockSpec((1, D), lambda i: (0, 0)),
        ],
        out_specs=pl.BlockSpec((ROWS_BLK, D), lambda i: (i, 0)),
        out_shape=jax.ShapeDtypeStruct((NPAD, D), jnp.float32),
    )(x_pad, W_emb, b_emb.reshape(1, D))


def _layer_norm_rows(hb, g, b):
    m = jnp.mean(hb, axis=-1, keepdims=True)
    v = jnp.mean((hb - m) ** 2, axis=-1, keepdims=True)
    return (hb - m) * lax.rsqrt(v + 1e-5) * g + b


def _qkv_body(h_ref, g_ref, b_ref, wq_ref, bq_ref, wk_ref, bk_ref,
              wv_ref, bv_ref, q_ref, k_ref, v_ref):
    hn = _layer_norm_rows(h_ref[...], g_ref[...], b_ref[...])
    scale = jnp.float32(1.0 / math.sqrt(D))
    q_ref[...] = (
        jnp.dot(hn, wq_ref[...], preferred_element_type=jnp.float32)
        + bq_ref[...]
    ) * scale
    k_ref[...] = (
        jnp.dot(hn, wk_ref[...], preferred_element_type=jnp.float32)
        + bk_ref[...]
    )
    v_ref[...] = (
        jnp.dot(hn, wv_ref[...], preferred_element_type=jnp.float32)
        + bv_ref[...]
    )


def _qkv(h, g, b, Wq, bq, Wk, bk, Wv, bv):
    row = lambda i: (i, 0)
    full = lambda i: (0, 0)
    return pl.pallas_call(
        _qkv_body,
        grid=(GRID_ROWS,),
        in_specs=[
            pl.BlockSpec((ROWS_BLK, D), row),
            pl.BlockSpec((1, D), full),
            pl.BlockSpec((1, D), full),
            pl.BlockSpec((D, D), full),
            pl.BlockSpec((1, D), full),
            pl.BlockSpec((D, D), full),
            pl.BlockSpec((1, D), full),
            pl.BlockSpec((D, D), full),
            pl.BlockSpec((1, D), full),
        ],
        out_specs=[
            pl.BlockSpec((ROWS_BLK, D), row),
            pl.BlockSpec((ROWS_BLK, D), row),
            pl.BlockSpec((ROWS_BLK, D), row),
        ],
        out_shape=[jax.ShapeDtypeStruct((NPAD, D), jnp.float32)] * 3,
    )(h, g.reshape(1, D), b.reshape(1, D), Wq, bq.reshape(1, D),
      Wk, bk.reshape(1, D), Wv, bv.reshape(1, D))


def _post_body(agg_ref, h_ref, wo_ref, bo_ref, g_ref, b_ref, o_ref):
    a = (
        jnp.dot(agg_ref[...], wo_ref[...], preferred_element_type=jnp.float32)
        + bo_ref[...]
    )
    o_ref[...] = h_ref[...] + jax.nn.relu(
        _layer_norm_rows(a, g_ref[...], b_ref[...]))


def _post(agg, h, Wo, bo, g, b):
    row = lambda i: (i, 0)
    full = lambda i: (0, 0)
    return pl.pallas_call(
        _post_body,
        grid=(GRID_ROWS,),
        in_specs=[
            pl.BlockSpec((ROWS_BLK, D), row),
            pl.BlockSpec((ROWS_BLK, D), row),
            pl.BlockSpec((D, D), full),
            pl.BlockSpec((1, D), full),
            pl.BlockSpec((1, D), full),
            pl.BlockSpec((1, D), full),
        ],
        out_specs=pl.BlockSpec((ROWS_BLK, D), row),
        out_shape=jax.ShapeDtypeStruct((NPAD, D), jnp.float32),
    )(agg, h, Wo, bo.reshape(1, D), g.reshape(1, D), b.reshape(1, D))


def _head_body(h_ref, w1_ref, b1_ref, w2_ref, b2_ref, o_ref):
    z = jax.nn.relu(
        jnp.dot(h_ref[...], w1_ref[...], preferred_element_type=jnp.float32)
        + b1_ref[...]
    )
    o_ref[...] = (
        jnp.dot(z, w2_ref[...], preferred_element_type=jnp.float32)
        + b2_ref[...]
    )


def _head(h, W1, b1, W2, b2):
    row = lambda i: (i, 0)
    full = lambda i: (0, 0)
    return pl.pallas_call(
        _head_body,
        grid=(GRID_ROWS,),
        in_specs=[
            pl.BlockSpec((ROWS_BLK, D), row),
            pl.BlockSpec((D, 2 * D), full),
            pl.BlockSpec((1, 2 * D), full),
            pl.BlockSpec((2 * D, C), full),
            pl.BlockSpec((1, C), full),
        ],
        out_specs=pl.BlockSpec((ROWS_BLK, C), row),
        out_shape=jax.ShapeDtypeStruct((NPAD, C), jnp.float32),
    )(h, W1, b1.reshape(1, 2 * D), W2, b2.reshape(1, C))


def _edge_phase(q, k, v, src, dst):
    scores = jnp.sum(q[dst] * k[src], axis=-1)
    e = jnp.exp(scores)
    denom = jax.ops.segment_sum(e, dst, num_segments=N)
    denom = jnp.where(denom > 0, denom, 1.0)
    attn = e / denom[dst]
    agg = jax.ops.segment_sum(attn[:, None] * v[src], dst, num_segments=N)
    return agg


def kernel(x, edge_index, W_emb, b_emb, ln1_g, ln1_b, ln2_g, ln2_b,
           Wq, bq, Wk, bk, Wv, bv, Wo, bo, W1, b1, W2, b2):
    src = edge_index[0]
    dst = edge_index[1]
    x_pad = jnp.pad(x, ((0, NPAD - N), (0, 0)))
    h = _embed(x_pad, W_emb, b_emb)
    for i in range(L):
        q, k, v = _qkv(h, ln1_g[i], ln1_b[i], Wq[i], bq[i],
                       Wk[i], bk[i], Wv[i], bv[i])
        agg = _edge_phase(q[:N], k[:N], v[:N], src, dst)
        agg = jnp.pad(agg, ((0, NPAD - N), (0, 0)))
        h = _post(agg, h, Wo[i], bo[i], ln2_g[i], ln2_b[i])
    logits = _head(h, W1, b1, W2, b2)
    return logits[:N]


# speedup vs baseline: 1.9966x; 1.5999x over previous
import functools
import math

import jax
import jax.numpy as jnp
from jax import lax
from jax.experimental import pallas as pl
from jax.experimental.pallas import tpu as pltpu
from jax.experimental.pallas import tpu_sc as plsc

N = 10000
E = 160000
D_IN = 256
D = 512
L = 4
C = 40

NPAD = 10240
ROWS_BLK = 1280
GRID_ROWS = NPAD // ROWS_BLK

SC_CORES = 2
SC_SUB = 16
NW = SC_CORES * SC_SUB
EPAD = 163840
EW1 = EPAD // NW
CH1 = 32
NCH1 = EW1 // CH1
EW2 = EPAD // SC_SUB
CH2 = 64
NCH2 = EW2 // CH2
STRIPE = NPAD // SC_SUB
DSLICE = 128
NSLICE = D // DSLICE


def _embed_body(x_ref, w_ref, b_ref, o_ref):
    o_ref[...] = (
        jnp.dot(x_ref[...], w_ref[...], preferred_element_type=jnp.float32)
        + b_ref[...]
    )


def _embed(x_pad, W_emb, b_emb):
    return pl.pallas_call(
        _embed_body,
        grid=(GRID_ROWS,),
        in_specs=[
            pl.BlockSpec((ROWS_BLK, D_IN), lambda i: (i, 0)),
            pl.BlockSpec((D_IN, D), lambda i: (0, 0)),
            pl.BlockSpec((1, D), lambda i: (0, 0)),
        ],
        out_specs=pl.BlockSpec((ROWS_BLK, D), lambda i: (i, 0)),
        out_shape=jax.ShapeDtypeStruct((NPAD, D), jnp.float32),
    )(x_pad, W_emb, b_emb.reshape(1, D))


def _layer_norm_rows(hb, g, b):
    m = jnp.mean(hb, axis=-1, keepdims=True)
    v = jnp.mean((hb - m) ** 2, axis=-1, keepdims=True)
    return (hb - m) * lax.rsqrt(v + 1e-5) * g + b


def _qkv_body(h_ref, g_ref, b_ref, wq_ref, bq_ref, wk_ref, bk_ref,
              wv_ref, bv_ref, q_ref, k_ref, v_ref):
    hn = _layer_norm_rows(h_ref[...], g_ref[...], b_ref[...])
    scale = jnp.float32(1.0 / math.sqrt(D))
    q_ref[...] = (
        jnp.dot(hn, wq_ref[...], preferred_element_type=jnp.float32)
        + bq_ref[...]
    ) * scale
    k_ref[...] = (
        jnp.dot(hn, wk_ref[...], preferred_element_type=jnp.float32)
        + bk_ref[...]
    )
    vmat = (
        jnp.dot(hn, wv_ref[...], preferred_element_type=jnp.float32)
        + bv_ref[...]
    )
    for j in range(NSLICE):
        v_ref[j] = vmat[:, j * DSLICE:(j + 1) * DSLICE]


def _qkv(h, g, b, Wq, bq, Wk, bk, Wv, bv):
    row = lambda i: (i, 0)
    full = lambda i: (0, 0)
    return pl.pallas_call(
        _qkv_body,
        grid=(GRID_ROWS,),
        in_specs=[
            pl.BlockSpec((ROWS_BLK, D), row),
            pl.BlockSpec((1, D), full),
            pl.BlockSpec((1, D), full),
            pl.BlockSpec((D, D), full),
            pl.BlockSpec((1, D), full),
            pl.BlockSpec((D, D), full),
            pl.BlockSpec((1, D), full),
            pl.BlockSpec((D, D), full),
            pl.BlockSpec((1, D), full),
        ],
        out_specs=[
            pl.BlockSpec((ROWS_BLK, D), row),
            pl.BlockSpec((ROWS_BLK, D), row),
            pl.BlockSpec((NSLICE, ROWS_BLK, DSLICE), lambda i: (0, i, 0)),
        ],
        out_shape=[
            jax.ShapeDtypeStruct((NPAD, D), jnp.float32),
            jax.ShapeDtypeStruct((NPAD, D), jnp.float32),
            jax.ShapeDtypeStruct((NSLICE, NPAD, DSLICE), jnp.float32),
        ],
    )(h, g.reshape(1, D), b.reshape(1, D), Wq, bq.reshape(1, D),
      Wk, bk.reshape(1, D), Wv, bv.reshape(1, D))


@functools.lru_cache(maxsize=None)
def _sc_mesh():
    return plsc.VectorSubcoreMesh(
        core_axis_name="c", subcore_axis_name="s",
        num_cores=SC_CORES, num_subcores=SC_SUB)


def _e1_body(q_hbm, k_hbm, src_hbm, dst_hbm, eatt_hbm, den2_hbm,
             src_big, dst_big, sidx, didx, qrows, krows, eatt_big, echunk,
             zstripe, den_sp):
    c = lax.axis_index("c")
    s = lax.axis_index("s")
    w = s * SC_CORES + c
    base = w * EW1
    for i in range(STRIPE // 16):
        zstripe[pl.ds(i * 16, 16)] = jnp.zeros((16,), jnp.float32)
    pltpu.sync_copy(zstripe, den_sp.at[pl.ds(s * STRIPE, STRIPE)])
    plsc.subcore_barrier()
    pltpu.sync_copy(src_hbm.at[pl.ds(base, EW1)], src_big)
    pltpu.sync_copy(dst_hbm.at[pl.ds(base, EW1)], dst_big)

    def chunk_body(ci, carry):
        off = ci * CH1
        for t in range(CH1 // 16):
            sidx[pl.ds(t * 16, 16)] = src_big[pl.ds(off + t * 16, 16)]
            didx[pl.ds(t * 16, 16)] = dst_big[pl.ds(off + t * 16, 16)]
        pltpu.sync_copy(q_hbm.at[didx], qrows)
        pltpu.sync_copy(k_hbm.at[sidx], krows)
        zero16 = jnp.zeros((16,), jnp.float32)
        for g in range(CH1 // 16):
            echunk[pl.ds(g * 16, 16)] = zero16
        for e in range(CH1):
            acc = qrows[e, pl.ds(0, 16)] * krows[e, pl.ds(0, 16)]
            for j in range(1, D // 16):
                acc = acc + (qrows[e, pl.ds(j * 16, 16)]
                             * krows[e, pl.ds(j * 16, 16)])
            plsc.addupdate_scatter(
                echunk, [jnp.full((16,), e, jnp.int32)], acc)
        for g in range(CH1 // 16):
            e16 = jnp.exp(echunk[pl.ds(g * 16, 16)])
            echunk[pl.ds(g * 16, 16)] = e16
            eatt_big[pl.ds(off + g * 16, 16)] = e16
        pltpu.sync_copy(echunk, den_sp.at[didx], add=True)
        return carry

    lax.fori_loop(0, NCH1, chunk_body, 0)
    pltpu.sync_copy(eatt_big, eatt_hbm.at[pl.ds(base, EW1)])
    plsc.subcore_barrier()
    pltpu.sync_copy(den_sp.at[pl.ds(s * STRIPE, STRIPE)],
                    den2_hbm.at[pl.ds(c * NPAD + s * STRIPE, STRIPE)])


@functools.lru_cache(maxsize=None)
def _e1():
    return pl.kernel(
        _e1_body,
        out_type=[
            jax.ShapeDtypeStruct((EPAD,), jnp.float32),
            jax.ShapeDtypeStruct((2 * NPAD,), jnp.float32),
        ],
        mesh=_sc_mesh(),
        compiler_params=pltpu.CompilerParams(
            use_tc_tiling_on_sc=False, needs_layout_passes=False),
        scratch_types=[
            pltpu.VMEM((EW1,), jnp.int32),
            pltpu.VMEM((EW1,), jnp.int32),
            pltpu.VMEM((CH1,), jnp.int32),
            pltpu.VMEM((CH1,), jnp.int32),
            pltpu.VMEM((CH1, D), jnp.float32),
            pltpu.VMEM((CH1, D), jnp.float32),
            pltpu.VMEM((EW1,), jnp.float32),
            pltpu.VMEM((CH1,), jnp.float32),
            pltpu.VMEM((STRIPE,), jnp.float32),
            pltpu.VMEM_SHARED((NPAD,), jnp.float32),
        ],
    )


def _e2_body(v_hbm, src_hbm, dst_hbm, eatt_hbm, den2_hbm, z_hbm, agg_hbm,
             src_big, dst_big, attn_big, echunk, gidx, didx, d0v, d1v,
             vrows, zrows, agg_sp):
    c = lax.axis_index("c")
    s = lax.axis_index("s")
    ebase = s * EW2
    pltpu.sync_copy(z_hbm, zrows)
    pltpu.sync_copy(src_hbm.at[pl.ds(ebase, EW2)], src_big)
    pltpu.sync_copy(dst_hbm.at[pl.ds(ebase, EW2)], dst_big)

    def attn_chunk(ci, carry):
        off = ci * CH2
        pltpu.sync_copy(eatt_hbm.at[pl.ds(ebase + off, CH2)], echunk)
        for t in range(CH2 // 16):
            gidx[pl.ds(t * 16, 16)] = dst_big[pl.ds(off + t * 16, 16)]
        pltpu.sync_copy(den2_hbm.at[gidx], d0v)
        for t in range(CH2 // 16):
            gidx[pl.ds(t * 16, 16)] = (
                dst_big[pl.ds(off + t * 16, 16)] + NPAD)
        pltpu.sync_copy(den2_hbm.at[gidx], d1v)
        for t in range(CH2 // 16):
            den = d0v[pl.ds(t * 16, 16)] + d1v[pl.ds(t * 16, 16)]
            den = jnp.where(den > 0.0, den, 1.0)
            attn_big[pl.ds(off + t * 16, 16)] = (
                echunk[pl.ds(t * 16, 16)] / den)
        return carry

    lax.fori_loop(0, NCH2, attn_chunk, 0)

    for rnd in range(NSLICE // SC_CORES):
        r = c * (NSLICE // SC_CORES) + rnd
        for zz in range(STRIPE // 16):
            pltpu.sync_copy(zrows,
                            agg_sp.at[pl.ds(s * STRIPE + zz * 16, 16)])
        plsc.subcore_barrier()

        def agg_chunk(ci, carry):
            off = ci * CH2
            rbase = r * NPAD
            for t in range(CH2 // 16):
                gidx[pl.ds(t * 16, 16)] = (
                    src_big[pl.ds(off + t * 16, 16)] + rbase)
                didx[pl.ds(t * 16, 16)] = dst_big[pl.ds(off + t * 16, 16)]
            pltpu.sync_copy(v_hbm.at[gidx], vrows)
            for t in range(CH2 // 16):
                a16 = attn_big[pl.ds(off + t * 16, 16)]
                for i in range(16):
                    e = t * 16 + i
                    av = jnp.full((16,), a16[i], jnp.float32)
                    for j in range(DSLICE // 16):
                        vrows[e, pl.ds(j * 16, 16)] = (
                            vrows[e, pl.ds(j * 16, 16)] * av)
            pltpu.sync_copy(vrows, agg_sp.at[didx], add=True)
            return carry

        lax.fori_loop(0, NCH2, agg_chunk, 0)
        plsc.subcore_barrier()
        pltpu.sync_copy(agg_sp.at[pl.ds(s * STRIPE, STRIPE)],
                        agg_hbm.at[pl.ds(r * NPAD + s * STRIPE, STRIPE)])


@functools.lru_cache(maxsize=None)
def _e2():
    return pl.kernel(
        _e2_body,
        out_type=jax.ShapeDtypeStruct((NSLICE * NPAD, DSLICE), jnp.float32),
        mesh=_sc_mesh(),
        compiler_params=pltpu.CompilerParams(
            use_tc_tiling_on_sc=False, needs_layout_passes=False),
        scratch_types=[
            pltpu.VMEM((EW2,), jnp.int32),
            pltpu.VMEM((EW2,), jnp.int32),
            pltpu.VMEM((EW2,), jnp.float32),
            pltpu.VMEM((CH2,), jnp.float32),
            pltpu.VMEM((CH2,), jnp.int32),
            pltpu.VMEM((CH2,), jnp.int32),
            pltpu.VMEM((CH2,), jnp.float32),
            pltpu.VMEM((CH2,), jnp.float32),
            pltpu.VMEM((CH2, DSLICE), jnp.float32),
            pltpu.VMEM((16, DSLICE), jnp.float32),
            pltpu.VMEM_SHARED((NPAD, DSLICE), jnp.float32),
        ],
    )


def _post_body(agg_ref, h_ref, wo_ref, bo_ref, g_ref, b_ref, o_ref):
    a = bo_ref[...]
    for j in range(NSLICE):
        a = a + jnp.dot(agg_ref[j], wo_ref[pl.ds(j * DSLICE, DSLICE), :],
                        preferred_element_type=jnp.float32)
    o_ref[...] = h_ref[...] + jax.nn.relu(
        _layer_norm_rows(a, g_ref[...], b_ref[...]))


def _post(agg4, h, Wo, bo, g, b):
    row = lambda i: (i, 0)
    full = lambda i: (0, 0)
    return pl.pallas_call(
        _post_body,
        grid=(GRID_ROWS,),
        in_specs=[
            pl.BlockSpec((NSLICE, ROWS_BLK, DSLICE), lambda i: (0, i, 0)),
            pl.BlockSpec((ROWS_BLK, D), row),
            pl.BlockSpec((D, D), full),
            pl.BlockSpec((1, D), full),
            pl.BlockSpec((1, D), full),
            pl.BlockSpec((1, D), full),
        ],
        out_specs=pl.BlockSpec((ROWS_BLK, D), row),
        out_shape=jax.ShapeDtypeStruct((NPAD, D), jnp.float32),
    )(agg4, h, Wo, bo.reshape(1, D), g.reshape(1, D), b.reshape(1, D))


def _head_body(h_ref, w1_ref, b1_ref, w2_ref, b2_ref, o_ref):
    z = jax.nn.relu(
        jnp.dot(h_ref[...], w1_ref[...], preferred_element_type=jnp.float32)
        + b1_ref[...]
    )
    o_ref[...] = (
        jnp.dot(z, w2_ref[...], preferred_element_type=jnp.float32)
        + b2_ref[...]
    )


def _head(h, W1, b1, W2, b2):
    row = lambda i: (i, 0)
    full = lambda i: (0, 0)
    return pl.pallas_call(
        _head_body,
        grid=(GRID_ROWS,),
        in_specs=[
            pl.BlockSpec((ROWS_BLK, D), row),
            pl.BlockSpec((D, 2 * D), full),
            pl.BlockSpec((1, 2 * D), full),
            pl.BlockSpec((2 * D, C), full),
            pl.BlockSpec((1, C), full),
        ],
        out_specs=pl.BlockSpec((ROWS_BLK, C), row),
        out_shape=jax.ShapeDtypeStruct((NPAD, C), jnp.float32),
    )(h, W1, b1.reshape(1, 2 * D), W2, b2.reshape(1, C))


def kernel(x, edge_index, W_emb, b_emb, ln1_g, ln1_b, ln2_g, ln2_b,
           Wq, bq, Wk, bk, Wv, bv, Wo, bo, W1, b1, W2, b2):
    pad = jnp.full((EPAD - E,), N, jnp.int32)
    src = jnp.concatenate([edge_index[0], pad])
    dst = jnp.concatenate([edge_index[1], pad])
    zrows = jnp.zeros((16, DSLICE), jnp.float32)
    x_pad = jnp.pad(x, ((0, NPAD - N), (0, 0)))
    h = _embed(x_pad, W_emb, b_emb)
    for i in range(L):
        q, k, v4 = _qkv(h, ln1_g[i], ln1_b[i], Wq[i], bq[i],
                        Wk[i], bk[i], Wv[i], bv[i])
        eatt, den2 = _e1()(q, k, src, dst)
        agg = _e2()(v4.reshape(NSLICE * NPAD, DSLICE), src, dst, eatt,
                    den2, zrows)
        h = _post(agg.reshape(NSLICE, NPAD, DSLICE), h,
                  Wo[i], bo[i], ln2_g[i], ln2_b[i])
    logits = _head(h, W1, b1, W2, b2)
    return logits[:N]


# speedup vs baseline: 3.4876x; 1.7468x over previous
import functools
import math

import jax
import jax.numpy as jnp
from jax import lax
from jax.experimental import pallas as pl
from jax.experimental.pallas import tpu as pltpu
from jax.experimental.pallas import tpu_sc as plsc

N = 10000
E = 160000
D_IN = 256
D = 512
L = 4
C = 40

NPAD = 10240
ROWS_BLK = 1280
GRID_ROWS = NPAD // ROWS_BLK

SC_CORES = 2
SC_SUB = 16
NW = SC_CORES * SC_SUB
EPAD = 163840
EW1 = EPAD // NW
CH1 = 16
NCH1 = EW1 // CH1
RING1 = 4
EW2 = EPAD // SC_SUB
CH2 = 64
NCH2 = EW2 // CH2
RING2 = 2
CH2A = 64
NCH2A = EW2 // CH2A
STRIPE = NPAD // SC_SUB
DSLICE = 128
NSLICE = D // DSLICE


def _embed_body(x_ref, w_ref, b_ref, o_ref):
    o_ref[...] = (
        jnp.dot(x_ref[...], w_ref[...], preferred_element_type=jnp.float32)
        + b_ref[...]
    )


def _embed(x_pad, W_emb, b_emb):
    return pl.pallas_call(
        _embed_body,
        grid=(GRID_ROWS,),
        in_specs=[
            pl.BlockSpec((ROWS_BLK, D_IN), lambda i: (i, 0)),
            pl.BlockSpec((D_IN, D), lambda i: (0, 0)),
            pl.BlockSpec((1, D), lambda i: (0, 0)),
        ],
        out_specs=pl.BlockSpec((ROWS_BLK, D), lambda i: (i, 0)),
        out_shape=jax.ShapeDtypeStruct((NPAD, D), jnp.float32),
    )(x_pad, W_emb, b_emb.reshape(1, D))


def _layer_norm_rows(hb, g, b):
    m = jnp.mean(hb, axis=-1, keepdims=True)
    v = jnp.mean((hb - m) ** 2, axis=-1, keepdims=True)
    return (hb - m) * lax.rsqrt(v + 1e-5) * g + b


def _qkv_body(h_ref, g_ref, b_ref, wq_ref, bq_ref, wk_ref, bk_ref,
              wv_ref, bv_ref, q_ref, k_ref, v_ref):
    hn = _layer_norm_rows(h_ref[...], g_ref[...], b_ref[...])
    scale = jnp.float32(1.0 / math.sqrt(D))
    q_ref[...] = ((
        jnp.dot(hn, wq_ref[...], preferred_element_type=jnp.float32)
        + bq_ref[...]
    ) * scale).astype(jnp.bfloat16)
    k_ref[...] = (
        jnp.dot(hn, wk_ref[...], preferred_element_type=jnp.float32)
        + bk_ref[...]
    ).astype(jnp.bfloat16)
    vmat = (
        jnp.dot(hn, wv_ref[...], preferred_element_type=jnp.float32)
        + bv_ref[...]
    )
    for j in range(NSLICE):
        v_ref[j] = vmat[:, j * DSLICE:(j + 1) * DSLICE]


def _qkv(h, g, b, Wq, bq, Wk, bk, Wv, bv):
    row = lambda i: (i, 0)
    full = lambda i: (0, 0)
    return pl.pallas_call(
        _qkv_body,
        grid=(GRID_ROWS,),
        in_specs=[
            pl.BlockSpec((ROWS_BLK, D), row),
            pl.BlockSpec((1, D), full),
            pl.BlockSpec((1, D), full),
            pl.BlockSpec((D, D), full),
            pl.BlockSpec((1, D), full),
            pl.BlockSpec((D, D), full),
            pl.BlockSpec((1, D), full),
            pl.BlockSpec((D, D), full),
            pl.BlockSpec((1, D), full),
        ],
        out_specs=[
            pl.BlockSpec((ROWS_BLK, D), row),
            pl.BlockSpec((ROWS_BLK, D), row),
            pl.BlockSpec((NSLICE, ROWS_BLK, DSLICE), lambda i: (0, i, 0)),
        ],
        out_shape=[
            jax.ShapeDtypeStruct((NPAD, D), jnp.bfloat16),
            jax.ShapeDtypeStruct((NPAD, D), jnp.bfloat16),
            jax.ShapeDtypeStruct((NSLICE, NPAD, DSLICE), jnp.float32),
        ],
    )(h, g.reshape(1, D), b.reshape(1, D), Wq, bq.reshape(1, D),
      Wk, bk.reshape(1, D), Wv, bv.reshape(1, D))


@functools.lru_cache(maxsize=None)
def _sc_mesh():
    return plsc.VectorSubcoreMesh(
        core_axis_name="c", subcore_axis_name="s",
        num_cores=SC_CORES, num_subcores=SC_SUB)


def _e1_body(q_hbm, k_hbm, src_hbm, dst_hbm, eatt_hbm, den2_hbm,
             src_big, dst_big, dst2, eatt_big,
             sidx0, sidx1, sidx2, sidx3, didx0, didx1, didx2, didx3,
             qr0, qr1, qr2, qr3, kr0, kr1, kr2, kr3,
             ech, zstripe,
             qs0, qs1, qs2, qs3, ks0, ks1, ks2, ks3, dsem,
             den_sp):
    sidx = [sidx0, sidx1, sidx2, sidx3]
    didx = [didx0, didx1, didx2, didx3]
    qr = [qr0, qr1, qr2, qr3]
    kr = [kr0, kr1, kr2, kr3]
    qs = [qs0, qs1, qs2, qs3]
    ks = [ks0, ks1, ks2, ks3]
    c = lax.axis_index("c")
    s = lax.axis_index("s")
    w = s * SC_CORES + c
    base = w * EW1
    for i in range(STRIPE // 16):
        zstripe[pl.ds(i * 16, 16)] = jnp.zeros((16,), jnp.float32)
    pltpu.sync_copy(zstripe, den_sp.at[pl.ds(s * STRIPE, STRIPE)])
    pltpu.sync_copy(src_hbm.at[pl.ds(base, EW1)], src_big)
    pltpu.sync_copy(dst_hbm.at[pl.ds(base, EW1)], dst_big)
    nrow2 = EW1 // 128
    d2descs = [
        pltpu.async_copy(dst_hbm.at[pl.ds(base + j * 128, 128)],
                         dst2.at[j], dsem)
        for j in range(nrow2)
    ]
    for dsc in d2descs:
        dsc.wait()

    def fire(b, ci):
        off = ci * CH1
        sidx[b][...] = src_big[pl.ds(off, CH1)]
        didx[b][...] = dst_big[pl.ds(off, CH1)]
        pltpu.async_copy(q_hbm.at[didx[b]], qr[b], qs[b])
        pltpu.async_copy(k_hbm.at[sidx[b]], kr[b], ks[b])

    for b in range(RING1):
        fire(b, b)

    def outer(ip, carry):
        for b in range(RING1):
            ci = ip * RING1 + b
            off = ci * CH1
            pltpu.make_async_copy(q_hbm.at[didx[b]], qr[b], qs[b]).wait()
            pltpu.make_async_copy(k_hbm.at[sidx[b]], kr[b], ks[b]).wait()
            ech[...] = jnp.zeros((16,), jnp.float32)
            for e in range(CH1):
                p = (qr[b][e, pl.ds(0, 32)] * kr[b][e, pl.ds(0, 32)])
                pa, pb = plsc.unpack(p, format=plsc.PackFormat.INTERLEAVED)
                acc = pa + pb
                for j in range(1, D // 32):
                    p = (qr[b][e, pl.ds(j * 32, 32)]
                         * kr[b][e, pl.ds(j * 32, 32)])
                    pa, pb = plsc.unpack(
                        p, format=plsc.PackFormat.INTERLEAVED)
                    acc = acc + pa + pb
                plsc.addupdate_scatter(
                    ech, [jnp.full((16,), e, jnp.int32)], acc)
            eatt_big[pl.ds(off, CH1)] = jnp.exp(ech[...])

            @pl.when(ci + RING1 < NCH1)
            def _():
                fire(b, ci + RING1)

        return carry

    lax.fori_loop(0, NCH1 // RING1, outer, 0)
    pltpu.sync_copy(eatt_big, eatt_hbm.at[pl.ds(base, EW1)])
    plsc.subcore_barrier()
    descs = []
    for j in range(nrow2):
        descs.append(pltpu.async_copy(
            eatt_big.at[pl.ds(j * 128, 128)],
            den_sp.at[dst2.at[j]], dsem, add=True))
    for dsc in descs:
        dsc.wait()
    plsc.subcore_barrier()
    pltpu.sync_copy(den_sp.at[pl.ds(s * STRIPE, STRIPE)],
                    den2_hbm.at[pl.ds(c * NPAD + s * STRIPE, STRIPE)])


@functools.lru_cache(maxsize=None)
def _e1():
    return pl.kernel(
        _e1_body,
        out_type=[
            jax.ShapeDtypeStruct((EPAD,), jnp.float32),
            jax.ShapeDtypeStruct((2 * NPAD,), jnp.float32),
        ],
        mesh=_sc_mesh(),
        compiler_params=pltpu.CompilerParams(
            use_tc_tiling_on_sc=False, needs_layout_passes=False),
        scratch_types=(
            [
                pltpu.VMEM((EW1,), jnp.int32),
                pltpu.VMEM((EW1,), jnp.int32),
                pltpu.VMEM((EW1 // 128, 128), jnp.int32),
                pltpu.VMEM((EW1,), jnp.float32),
            ]
            + [pltpu.VMEM((CH1,), jnp.int32)] * 8
            + [pltpu.VMEM((CH1, D), jnp.bfloat16)] * 8
            + [
                pltpu.VMEM((CH1,), jnp.float32),
                pltpu.VMEM((STRIPE,), jnp.float32),
            ]
            + [pltpu.SemaphoreType.DMA] * 9
            + [pltpu.VMEM_SHARED((NPAD,), jnp.float32)]
        ),
    )


def _e2_body(v_hbm, src_hbm, dst_hbm, eatt_hbm, den2_hbm, z_hbm, agg_hbm,
             src_big, dst_big, attn_big,
             ea0, ea1, d00, d01, d10, d11, g00, g01, g10, g11,
             gv0, gv1, sd0, sd1, vr0, vr1,
             es0, es1, ds00, ds01, ds10, ds11, vs0, vs1,
             zrows, agg_sp):
    ea = [ea0, ea1]
    d0 = [d00, d01]
    d1 = [d10, d11]
    g0 = [g00, g01]
    g1 = [g10, g11]
    es = [es0, es1]
    ds0 = [ds00, ds01]
    ds1 = [ds10, ds11]
    gv = [gv0, gv1]
    sd = [sd0, sd1]
    vr = [vr0, vr1]
    vs = [vs0, vs1]
    c = lax.axis_index("c")
    s = lax.axis_index("s")
    ebase = s * EW2
    pltpu.sync_copy(z_hbm, zrows)
    pltpu.sync_copy(src_hbm.at[pl.ds(ebase, EW2)], src_big)
    pltpu.sync_copy(dst_hbm.at[pl.ds(ebase, EW2)], dst_big)

    def fire_attn(b, ci):
        off = ci * CH2A
        for t in range(CH2A // 16):
            g0[b][pl.ds(t * 16, 16)] = dst_big[pl.ds(off + t * 16, 16)]
            g1[b][pl.ds(t * 16, 16)] = (
                dst_big[pl.ds(off + t * 16, 16)] + NPAD)
        pltpu.async_copy(eatt_hbm.at[pl.ds(ebase + off, CH2A)],
                         ea[b], es[b])
        pltpu.async_copy(den2_hbm.at[g0[b]], d0[b], ds0[b])
        pltpu.async_copy(den2_hbm.at[g1[b]], d1[b], ds1[b])

    for b in range(RING2):
        fire_attn(b, b)

    def attn_loop(ip, carry):
        for b in range(RING2):
            ci = ip * RING2 + b
            off = ci * CH2A
            pltpu.make_async_copy(
                eatt_hbm.at[pl.ds(ebase + off, CH2A)], ea[b],
                es[b]).wait()
            pltpu.make_async_copy(den2_hbm.at[g0[b]], d0[b],
                                  ds0[b]).wait()
            pltpu.make_async_copy(den2_hbm.at[g1[b]], d1[b],
                                  ds1[b]).wait()
            for t in range(CH2A // 16):
                den = d0[b][pl.ds(t * 16, 16)] + d1[b][pl.ds(t * 16, 16)]
                den = jnp.where(den > 0.0, den, 1.0)
                attn_big[pl.ds(off + t * 16, 16)] = (
                    ea[b][pl.ds(t * 16, 16)] / den)

            @pl.when(ci + RING2 < NCH2A)
            def _():
                fire_attn(b, ci + RING2)

        return carry

    lax.fori_loop(0, NCH2A // RING2, attn_loop, 0)

    for rnd in range(NSLICE // SC_CORES):
        r = c * (NSLICE // SC_CORES) + rnd
        rbase = r * NPAD
        for zz in range(STRIPE // 8):
            pltpu.sync_copy(zrows,
                            agg_sp.at[pl.ds(s * STRIPE + zz * 8, 8)])
        plsc.subcore_barrier()

        def fire_v(b, ci):
            off = ci * CH2
            for t in range(CH2 // 16):
                gv[b][pl.ds(t * 16, 16)] = (
                    src_big[pl.ds(off + t * 16, 16)] + rbase)
                sd[b][pl.ds(t * 16, 16)] = dst_big[pl.ds(off + t * 16, 16)]
            pltpu.async_copy(v_hbm.at[gv[b]], vr[b], vs[b])

        for b in range(RING2):
            fire_v(b, b)

        def agg_loop(ip, carry):
            for b in range(RING2):
                ci = ip * RING2 + b
                off = ci * CH2
                pltpu.make_async_copy(v_hbm.at[gv[b]], vr[b],
                                      vs[b]).wait()
                for t in range(CH2 // 16):
                    a16 = attn_big[pl.ds(off + t * 16, 16)]
                    for i in range(16):
                        e = t * 16 + i
                        av = jnp.full((16,), a16[i], jnp.float32)
                        for j in range(DSLICE // 16):
                            vr[b][e, pl.ds(j * 16, 16)] = (
                                vr[b][e, pl.ds(j * 16, 16)] * av)
                pltpu.sync_copy(vr[b], agg_sp.at[sd[b]], add=True)

                @pl.when(ci + RING2 < NCH2)
                def _():
                    fire_v(b, ci + RING2)

            return carry

        lax.fori_loop(0, NCH2 // RING2, agg_loop, 0)
        plsc.subcore_barrier()
        pltpu.sync_copy(agg_sp.at[pl.ds(s * STRIPE, STRIPE)],
                        agg_hbm.at[pl.ds(rbase + s * STRIPE, STRIPE)])


@functools.lru_cache(maxsize=None)
def _e2():
    return pl.kernel(
        _e2_body,
        out_type=jax.ShapeDtypeStruct((NSLICE * NPAD, DSLICE), jnp.float32),
        mesh=_sc_mesh(),
        compiler_params=pltpu.CompilerParams(
            use_tc_tiling_on_sc=False, needs_layout_passes=False),
        scratch_types=(
            [
                pltpu.VMEM((EW2,), jnp.int32),
                pltpu.VMEM((EW2,), jnp.int32),
                pltpu.VMEM((EW2,), jnp.float32),
            ]
            + [pltpu.VMEM((CH2A,), jnp.float32)] * 6
            + [pltpu.VMEM((CH2A,), jnp.int32)] * 4
            + [pltpu.VMEM((CH2,), jnp.int32)] * 4
            + [pltpu.VMEM((CH2, DSLICE), jnp.float32)] * 2
            + [pltpu.SemaphoreType.DMA] * 8
            + [
                pltpu.VMEM((8, DSLICE), jnp.float32),
                pltpu.VMEM_SHARED((NPAD, DSLICE), jnp.float32),
            ]
        ),
    )


def _post_body(agg_ref, h_ref, wo_ref, bo_ref, g_ref, b_ref, o_ref):
    a = bo_ref[...]
    for j in range(NSLICE):
        a = a + jnp.dot(agg_ref[j], wo_ref[pl.ds(j * DSLICE, DSLICE), :],
                        preferred_element_type=jnp.float32)
    o_ref[...] = h_ref[...] + jax.nn.relu(
        _layer_norm_rows(a, g_ref[...], b_ref[...]))


def _post(agg4, h, Wo, bo, g, b):
    row = lambda i: (i, 0)
    full = lambda i: (0, 0)
    return pl.pallas_call(
        _post_body,
        grid=(GRID_ROWS,),
        in_specs=[
            pl.BlockSpec((NSLICE, ROWS_BLK, DSLICE), lambda i: (0, i, 0)),
            pl.BlockSpec((ROWS_BLK, D), row),
            pl.BlockSpec((D, D), full),
            pl.BlockSpec((1, D), full),
            pl.BlockSpec((1, D), full),
            pl.BlockSpec((1, D), full),
        ],
        out_specs=pl.BlockSpec((ROWS_BLK, D), row),
        out_shape=jax.ShapeDtypeStruct((NPAD, D), jnp.float32),
    )(agg4, h, Wo, bo.reshape(1, D), g.reshape(1, D), b.reshape(1, D))


def _head_body(h_ref, w1_ref, b1_ref, w2_ref, b2_ref, o_ref):
    z = jax.nn.relu(
        jnp.dot(h_ref[...], w1_ref[...], preferred_element_type=jnp.float32)
        + b1_ref[...]
    )
    o_ref[...] = (
        jnp.dot(z, w2_ref[...], preferred_element_type=jnp.float32)
        + b2_ref[...]
    )


def _head(h, W1, b1, W2, b2):
    row = lambda i: (i, 0)
    full = lambda i: (0, 0)
    return pl.pallas_call(
        _head_body,
        grid=(GRID_ROWS,),
        in_specs=[
            pl.BlockSpec((ROWS_BLK, D), row),
            pl.BlockSpec((D, 2 * D), full),
            pl.BlockSpec((1, 2 * D), full),
            pl.BlockSpec((2 * D, C), full),
            pl.BlockSpec((1, C), full),
        ],
        out_specs=pl.BlockSpec((ROWS_BLK, C), row),
        out_shape=jax.ShapeDtypeStruct((NPAD, C), jnp.float32),
    )(h, W1, b1.reshape(1, 2 * D), W2, b2.reshape(1, C))


def kernel(x, edge_index, W_emb, b_emb, ln1_g, ln1_b, ln2_g, ln2_b,
           Wq, bq, Wk, bk, Wv, bv, Wo, bo, W1, b1, W2, b2):
    pad = jnp.full((EPAD - E,), N, jnp.int32)
    src = jnp.concatenate([edge_index[0], pad])
    dst = jnp.concatenate([edge_index[1], pad])
    zrows = jnp.zeros((8, DSLICE), jnp.float32)
    x_pad = jnp.pad(x, ((0, NPAD - N), (0, 0)))
    h = _embed(x_pad, W_emb, b_emb)
    for i in range(L):
        q, k, v4 = _qkv(h, ln1_g[i], ln1_b[i], Wq[i], bq[i],
                        Wk[i], bk[i], Wv[i], bv[i])
        eatt, den2 = _e1()(q, k, src, dst)
        agg = _e2()(v4.reshape(NSLICE * NPAD, DSLICE), src, dst, eatt,
                    den2, zrows)
        h = _post(agg.reshape(NSLICE, NPAD, DSLICE), h,
                  Wo[i], bo[i], ln2_g[i], ln2_b[i])
    logits = _head(h, W1, b1, W2, b2)
    return logits[:N]


# speedup vs baseline: 3.6697x; 1.0522x over previous
import functools
import math

import jax
import jax.numpy as jnp
from jax import lax
from jax.experimental import pallas as pl
from jax.experimental.pallas import tpu as pltpu
from jax.experimental.pallas import tpu_sc as plsc

N = 10000
E = 160000
D_IN = 256
D = 512
L = 4
C = 40

NPAD = 10240
ROWS_BLK = 1280
GRID_ROWS = NPAD // ROWS_BLK

SC_CORES = 2
SC_SUB = 16
NW = SC_CORES * SC_SUB
EPAD = 163840
EW1 = EPAD // NW
CH1 = 32
NCH1 = EW1 // CH1
RING1 = 2
EW2 = EPAD // SC_SUB
CH2 = 80
NCH2 = EW2 // CH2
RING2 = 2
STRIPE = NPAD // SC_SUB
DSLICE = 128
NSLICE = D // DSLICE


def _embed_body(x_ref, w_ref, b_ref, o_ref):
    o_ref[...] = (
        jnp.dot(x_ref[...], w_ref[...], preferred_element_type=jnp.float32)
        + b_ref[...]
    )


def _embed(x_pad, W_emb, b_emb):
    return pl.pallas_call(
        _embed_body,
        grid=(GRID_ROWS,),
        in_specs=[
            pl.BlockSpec((ROWS_BLK, D_IN), lambda i: (i, 0)),
            pl.BlockSpec((D_IN, D), lambda i: (0, 0)),
            pl.BlockSpec((1, D), lambda i: (0, 0)),
        ],
        out_specs=pl.BlockSpec((ROWS_BLK, D), lambda i: (i, 0)),
        out_shape=jax.ShapeDtypeStruct((NPAD, D), jnp.float32),
    )(x_pad, W_emb, b_emb.reshape(1, D))


def _layer_norm_rows(hb, g, b):
    m = jnp.mean(hb, axis=-1, keepdims=True)
    v = jnp.mean((hb - m) ** 2, axis=-1, keepdims=True)
    return (hb - m) * lax.rsqrt(v + 1e-5) * g + b


def _qkv_body(h_ref, g_ref, b_ref, wq_ref, bq_ref, wk_ref, bk_ref,
              wv_ref, bv_ref, q_ref, k_ref, v_ref):
    hn = _layer_norm_rows(h_ref[...], g_ref[...], b_ref[...])
    scale = jnp.float32(1.0 / math.sqrt(D))
    q_ref[...] = ((
        jnp.dot(hn, wq_ref[...], preferred_element_type=jnp.float32)
        + bq_ref[...]
    ) * scale).astype(jnp.bfloat16)
    k_ref[...] = (
        jnp.dot(hn, wk_ref[...], preferred_element_type=jnp.float32)
        + bk_ref[...]
    ).astype(jnp.bfloat16)
    vmat = (
        jnp.dot(hn, wv_ref[...], preferred_element_type=jnp.float32)
        + bv_ref[...]
    )
    for j in range(NSLICE):
        v_ref[j] = vmat[:, j * DSLICE:(j + 1) * DSLICE]


def _qkv(h, g, b, Wq, bq, Wk, bk, Wv, bv):
    row = lambda i: (i, 0)
    full = lambda i: (0, 0)
    return pl.pallas_call(
        _qkv_body,
        grid=(GRID_ROWS,),
        in_specs=[
            pl.BlockSpec((ROWS_BLK, D), row),
            pl.BlockSpec((1, D), full),
            pl.BlockSpec((1, D), full),
            pl.BlockSpec((D, D), full),
            pl.BlockSpec((1, D), full),
            pl.BlockSpec((D, D), full),
            pl.BlockSpec((1, D), full),
            pl.BlockSpec((D, D), full),
            pl.BlockSpec((1, D), full),
        ],
        out_specs=[
            pl.BlockSpec((ROWS_BLK, D), row),
            pl.BlockSpec((ROWS_BLK, D), row),
            pl.BlockSpec((NSLICE, ROWS_BLK, DSLICE), lambda i: (0, i, 0)),
        ],
        out_shape=[
            jax.ShapeDtypeStruct((NPAD, D), jnp.bfloat16),
            jax.ShapeDtypeStruct((NPAD, D), jnp.bfloat16),
            jax.ShapeDtypeStruct((NSLICE, NPAD, DSLICE), jnp.float32),
        ],
    )(h, g.reshape(1, D), b.reshape(1, D), Wq, bq.reshape(1, D),
      Wk, bk.reshape(1, D), Wv, bv.reshape(1, D))


@functools.lru_cache(maxsize=None)
def _sc_mesh():
    return plsc.VectorSubcoreMesh(
        core_axis_name="c", subcore_axis_name="s",
        num_cores=SC_CORES, num_subcores=SC_SUB)


def _e1_body(q_hbm, k_hbm, src_hbm, dst_hbm, eatt_hbm, den2_hbm,
             src_big, dst_big, dst2, eatt_big,
             sidx0, sidx1, didx0, didx1,
             qr0, qr1, kr0, kr1,
             ech, zstripe,
             qs0, qs1, ks0, ks1, dsem,
             den_sp):
    sidx = [sidx0, sidx1]
    didx = [didx0, didx1]
    qr = [qr0, qr1]
    kr = [kr0, kr1]
    qs = [qs0, qs1]
    ks = [ks0, ks1]
    c = lax.axis_index("c")
    s = lax.axis_index("s")
    w = s * SC_CORES + c
    base = w * EW1
    for i in range(STRIPE // 16):
        zstripe[pl.ds(i * 16, 16)] = jnp.zeros((16,), jnp.float32)
    pltpu.sync_copy(zstripe, den_sp.at[pl.ds(s * STRIPE, STRIPE)])
    pltpu.sync_copy(src_hbm.at[pl.ds(base, EW1)], src_big)
    pltpu.sync_copy(dst_hbm.at[pl.ds(base, EW1)], dst_big)
    nrow2 = EW1 // 128
    d2descs = [
        pltpu.async_copy(dst_hbm.at[pl.ds(base + j * 128, 128)],
                         dst2.at[j], dsem)
        for j in range(nrow2)
    ]
    for dsc in d2descs:
        dsc.wait()

    def fire(b, ci):
        off = ci * CH1
        for t in range(CH1 // 16):
            sidx[b][pl.ds(t * 16, 16)] = src_big[pl.ds(off + t * 16, 16)]
            didx[b][pl.ds(t * 16, 16)] = dst_big[pl.ds(off + t * 16, 16)]
        pltpu.async_copy(q_hbm.at[didx[b]], qr[b], qs[b])
        pltpu.async_copy(k_hbm.at[sidx[b]], kr[b], ks[b])

    for b in range(RING1):
        fire(b, b)

    def outer(ip, carry):
        for b in range(RING1):
            ci = ip * RING1 + b
            off = ci * CH1
            pltpu.make_async_copy(q_hbm.at[didx[b]], qr[b], qs[b]).wait()
            pltpu.make_async_copy(k_hbm.at[sidx[b]], kr[b], ks[b]).wait()
            for g in range(CH1 // 16):
                ech[pl.ds(g * 16, 16)] = jnp.zeros((16,), jnp.float32)
            for e in range(CH1):
                p = (qr[b][e, pl.ds(0, 32)] * kr[b][e, pl.ds(0, 32)])
                pa, pb = plsc.unpack(p, format=plsc.PackFormat.INTERLEAVED)
                acc = pa + pb
                for j in range(1, D // 32):
                    p = (qr[b][e, pl.ds(j * 32, 32)]
                         * kr[b][e, pl.ds(j * 32, 32)])
                    pa, pb = plsc.unpack(
                        p, format=plsc.PackFormat.INTERLEAVED)
                    acc = acc + pa + pb
                plsc.addupdate_scatter(
                    ech, [jnp.full((16,), e, jnp.int32)], acc)
            for g in range(CH1 // 16):
                eatt_big[pl.ds(off + g * 16, 16)] = jnp.exp(
                    ech[pl.ds(g * 16, 16)])

            @pl.when(ci + RING1 < NCH1)
            def _():
                fire(b, ci + RING1)

        return carry

    lax.fori_loop(0, NCH1 // RING1, outer, 0)
    pltpu.sync_copy(eatt_big, eatt_hbm.at[pl.ds(base, EW1)])
    plsc.subcore_barrier()
    descs = []
    for j in range(nrow2):
        descs.append(pltpu.async_copy(
            eatt_big.at[pl.ds(j * 128, 128)],
            den_sp.at[dst2.at[j]], dsem, add=True))
    for dsc in descs:
        dsc.wait()
    plsc.subcore_barrier()
    pltpu.sync_copy(den_sp.at[pl.ds(s * STRIPE, STRIPE)],
                    den2_hbm.at[pl.ds(c * NPAD + s * STRIPE, STRIPE)])


@functools.lru_cache(maxsize=None)
def _e1():
    return pl.kernel(
        _e1_body,
        out_type=[
            jax.ShapeDtypeStruct((EPAD,), jnp.float32),
            jax.ShapeDtypeStruct((2 * NPAD,), jnp.float32),
        ],
        mesh=_sc_mesh(),
        compiler_params=pltpu.CompilerParams(
            use_tc_tiling_on_sc=False, needs_layout_passes=False),
        scratch_types=(
            [
                pltpu.VMEM((EW1,), jnp.int32),
                pltpu.VMEM((EW1,), jnp.int32),
                pltpu.VMEM((EW1 // 128, 128), jnp.int32),
                pltpu.VMEM((EW1,), jnp.float32),
            ]
            + [pltpu.VMEM((CH1,), jnp.int32)] * 4
            + [pltpu.VMEM((CH1, D), jnp.bfloat16)] * 4
            + [
                pltpu.VMEM((CH1,), jnp.float32),
                pltpu.VMEM((STRIPE,), jnp.float32),
            ]
            + [pltpu.SemaphoreType.DMA] * 5
            + [pltpu.VMEM_SHARED((NPAD,), jnp.float32)]
        ),
    )


def _e2_body(v_hbm, src_hbm, dst_hbm, eatt_hbm, z_hbm, agg_hbm,
             src_big, dst_big,
             ea0, ea1, gv0, gv1, sd0, sd1, vr0, vr1,
             es0, es1, vs0, vs1,
             zrows, agg_sp):
    ea = [ea0, ea1]
    es = [es0, es1]
    gv = [gv0, gv1]
    sd = [sd0, sd1]
    vr = [vr0, vr1]
    vs = [vs0, vs1]
    c = lax.axis_index("c")
    s = lax.axis_index("s")
    ebase = s * EW2
    pltpu.sync_copy(z_hbm, zrows)
    pltpu.sync_copy(src_hbm.at[pl.ds(ebase, EW2)], src_big)
    pltpu.sync_copy(dst_hbm.at[pl.ds(ebase, EW2)], dst_big)

    for rnd in range(NSLICE // SC_CORES):
        r = c * (NSLICE // SC_CORES) + rnd
        rbase = r * NPAD
        for zz in range(STRIPE // 8):
            pltpu.sync_copy(zrows,
                            agg_sp.at[pl.ds(s * STRIPE + zz * 8, 8)])
        plsc.subcore_barrier()

        def fire_v(b, ci):
            off = ci * CH2
            for t in range(CH2 // 16):
                gv[b][pl.ds(t * 16, 16)] = (
                    src_big[pl.ds(off + t * 16, 16)] + rbase)
                sd[b][pl.ds(t * 16, 16)] = dst_big[pl.ds(off + t * 16, 16)]
            pltpu.async_copy(v_hbm.at[gv[b]], vr[b], vs[b])
            pltpu.async_copy(eatt_hbm.at[pl.ds(ebase + off, CH2)],
                             ea[b], es[b])

        for b in range(RING2):
            fire_v(b, b)

        def agg_loop(ip, carry):
            for b in range(RING2):
                ci = ip * RING2 + b
                off = ci * CH2
                pltpu.make_async_copy(v_hbm.at[gv[b]], vr[b],
                                      vs[b]).wait()
                pltpu.make_async_copy(
                    eatt_hbm.at[pl.ds(ebase + off, CH2)], ea[b],
                    es[b]).wait()
                for t in range(CH2 // 16):
                    a16 = ea[b][pl.ds(t * 16, 16)]
                    for i in range(16):
                        e = t * 16 + i
                        av = jnp.full((16,), a16[i], jnp.float32)
                        for j in range(DSLICE // 16):
                            vr[b][e, pl.ds(j * 16, 16)] = (
                                vr[b][e, pl.ds(j * 16, 16)] * av)
                pltpu.sync_copy(vr[b], agg_sp.at[sd[b]], add=True)

                @pl.when(ci + RING2 < NCH2)
                def _():
                    fire_v(b, ci + RING2)

            return carry

        lax.fori_loop(0, NCH2 // RING2, agg_loop, 0)
        plsc.subcore_barrier()
        pltpu.sync_copy(agg_sp.at[pl.ds(s * STRIPE, STRIPE)],
                        agg_hbm.at[pl.ds(rbase + s * STRIPE, STRIPE)])


@functools.lru_cache(maxsize=None)
def _e2():
    return pl.kernel(
        _e2_body,
        out_type=jax.ShapeDtypeStruct((NSLICE * NPAD, DSLICE), jnp.float32),
        mesh=_sc_mesh(),
        compiler_params=pltpu.CompilerParams(
            use_tc_tiling_on_sc=False, needs_layout_passes=False),
        scratch_types=(
            [
                pltpu.VMEM((EW2,), jnp.int32),
                pltpu.VMEM((EW2,), jnp.int32),
            ]
            + [pltpu.VMEM((CH2,), jnp.float32)] * 2
            + [pltpu.VMEM((CH2,), jnp.int32)] * 4
            + [pltpu.VMEM((CH2, DSLICE), jnp.float32)] * 2
            + [pltpu.SemaphoreType.DMA] * 4
            + [
                pltpu.VMEM((8, DSLICE), jnp.float32),
                pltpu.VMEM_SHARED((NPAD, DSLICE), jnp.float32),
            ]
        ),
    )


def _post_body(agg_ref, den_ref, h_ref, wo_ref, bo_ref, g_ref, b_ref,
               o_ref):
    acc = jnp.dot(agg_ref[0], wo_ref[pl.ds(0, DSLICE), :],
                  preferred_element_type=jnp.float32)
    for j in range(1, NSLICE):
        acc = acc + jnp.dot(agg_ref[j],
                            wo_ref[pl.ds(j * DSLICE, DSLICE), :],
                            preferred_element_type=jnp.float32)
    den = den_ref[:, 0:1] + den_ref[:, 1:2]
    inv = jnp.where(den > 0.0, 1.0 / den, 1.0)
    a = acc * inv + bo_ref[...]
    o_ref[...] = h_ref[...] + jax.nn.relu(
        _layer_norm_rows(a, g_ref[...], b_ref[...]))


def _post(agg4, den2, h, Wo, bo, g, b):
    row = lambda i: (i, 0)
    full = lambda i: (0, 0)
    return pl.pallas_call(
        _post_body,
        grid=(GRID_ROWS,),
        in_specs=[
            pl.BlockSpec((NSLICE, ROWS_BLK, DSLICE), lambda i: (0, i, 0)),
            pl.BlockSpec((ROWS_BLK, 2), lambda i: (i, 0)),
            pl.BlockSpec((ROWS_BLK, D), row),
            pl.BlockSpec((D, D), full),
            pl.BlockSpec((1, D), full),
            pl.BlockSpec((1, D), full),
            pl.BlockSpec((1, D), full),
        ],
        out_specs=pl.BlockSpec((ROWS_BLK, D), row),
        out_shape=jax.ShapeDtypeStruct((NPAD, D), jnp.float32),
    )(agg4, den2, h, Wo, bo.reshape(1, D), g.reshape(1, D),
      b.reshape(1, D))


def _head_body(h_ref, w1_ref, b1_ref, w2_ref, b2_ref, o_ref):
    z = jax.nn.relu(
        jnp.dot(h_ref[...], w1_ref[...], preferred_element_type=jnp.float32)
        + b1_ref[...]
    )
    o_ref[...] = (
        jnp.dot(z, w2_ref[...], preferred_element_type=jnp.float32)
        + b2_ref[...]
    )


def _head(h, W1, b1, W2, b2):
    row = lambda i: (i, 0)
    full = lambda i: (0, 0)
    return pl.pallas_call(
        _head_body,
        grid=(GRID_ROWS,),
        in_specs=[
            pl.BlockSpec((ROWS_BLK, D), row),
            pl.BlockSpec((D, 2 * D), full),
            pl.BlockSpec((1, 2 * D), full),
            pl.BlockSpec((2 * D, C), full),
            pl.BlockSpec((1, C), full),
        ],
        out_specs=pl.BlockSpec((ROWS_BLK, C), row),
        out_shape=jax.ShapeDtypeStruct((NPAD, C), jnp.float32),
    )(h, W1, b1.reshape(1, 2 * D), W2, b2.reshape(1, C))


def kernel(x, edge_index, W_emb, b_emb, ln1_g, ln1_b, ln2_g, ln2_b,
           Wq, bq, Wk, bk, Wv, bv, Wo, bo, W1, b1, W2, b2):
    pad = jnp.full((EPAD - E,), N, jnp.int32)
    src = jnp.concatenate([edge_index[0], pad])
    dst = jnp.concatenate([edge_index[1], pad])
    zrows = jnp.zeros((8, DSLICE), jnp.float32)
    x_pad = jnp.pad(x, ((0, NPAD - N), (0, 0)))
    h = _embed(x_pad, W_emb, b_emb)
    for i in range(L):
        q, k, v4 = _qkv(h, ln1_g[i], ln1_b[i], Wq[i], bq[i],
                        Wk[i], bk[i], Wv[i], bv[i])
        eatt, den2 = _e1()(q, k, src, dst)
        agg = _e2()(v4.reshape(NSLICE * NPAD, DSLICE), src, dst, eatt,
                    zrows)
        den2t = den2.reshape(2, NPAD).T
        h = _post(agg.reshape(NSLICE, NPAD, DSLICE), den2t, h,
                  Wo[i], bo[i], ln2_g[i], ln2_b[i])
    logits = _head(h, W1, b1, W2, b2)
    return logits[:N]


# speedup vs baseline: 4.2778x; 1.1657x over previous
import functools
import math

import jax
import jax.numpy as jnp
from jax import lax
from jax.experimental import pallas as pl
from jax.experimental.pallas import tpu as pltpu
from jax.experimental.pallas import tpu_sc as plsc

N = 10000
E = 160000
D_IN = 256
D = 512
L = 4
C = 40

NPAD = 10240
ROWS_BLK = 1280
GRID_ROWS = NPAD // ROWS_BLK

SC_CORES = 2
SC_SUB = 16
NW = SC_CORES * SC_SUB
EPAD = 163840
EW1 = EPAD // NW
CH1 = 32
NCH1 = EW1 // CH1
RING1 = 2
EW2 = EPAD // SC_SUB
CH2 = 80
NCH2 = EW2 // CH2
RING2 = 2
STRIPE = NPAD // SC_SUB
DSLICE = 128
NSLICE = D // DSLICE


def _embed_body(x_ref, w_ref, b_ref, o_ref):
    o_ref[...] = (
        jnp.dot(x_ref[...], w_ref[...], preferred_element_type=jnp.float32)
        + b_ref[...]
    )


def _embed(x_pad, W_emb, b_emb):
    return pl.pallas_call(
        _embed_body,
        grid=(GRID_ROWS,),
        in_specs=[
            pl.BlockSpec((ROWS_BLK, D_IN), lambda i: (i, 0)),
            pl.BlockSpec((D_IN, D), lambda i: (0, 0)),
            pl.BlockSpec((1, D), lambda i: (0, 0)),
        ],
        out_specs=pl.BlockSpec((ROWS_BLK, D), lambda i: (i, 0)),
        out_shape=jax.ShapeDtypeStruct((NPAD, D), jnp.float32),
    )(x_pad, W_emb, b_emb.reshape(1, D))


def _layer_norm_rows(hb, g, b):
    m = jnp.mean(hb, axis=-1, keepdims=True)
    v = jnp.mean((hb - m) ** 2, axis=-1, keepdims=True)
    return (hb - m) * lax.rsqrt(v + 1e-5) * g + b


def _qkv_body(h_ref, g_ref, b_ref, wq_ref, bq_ref, wk_ref, bk_ref,
              wv_ref, bv_ref, q_ref, k_ref, v_ref):
    hn = _layer_norm_rows(h_ref[...], g_ref[...], b_ref[...])
    scale = jnp.float32(1.0 / math.sqrt(D))
    q_ref[...] = ((
        jnp.dot(hn, wq_ref[...], preferred_element_type=jnp.float32)
        + bq_ref[...]
    ) * scale).astype(jnp.bfloat16)
    k_ref[...] = (
        jnp.dot(hn, wk_ref[...], preferred_element_type=jnp.float32)
        + bk_ref[...]
    ).astype(jnp.bfloat16)
    vmat = (
        jnp.dot(hn, wv_ref[...], preferred_element_type=jnp.float32)
        + bv_ref[...]
    )
    for j in range(NSLICE):
        v_ref[j] = vmat[:, j * DSLICE:(j + 1) * DSLICE].astype(jnp.bfloat16)


def _qkv(h, g, b, Wq, bq, Wk, bk, Wv, bv):
    row = lambda i: (i, 0)
    full = lambda i: (0, 0)
    return pl.pallas_call(
        _qkv_body,
        grid=(GRID_ROWS,),
        in_specs=[
            pl.BlockSpec((ROWS_BLK, D), row),
            pl.BlockSpec((1, D), full),
            pl.BlockSpec((1, D), full),
            pl.BlockSpec((D, D), full),
            pl.BlockSpec((1, D), full),
            pl.BlockSpec((D, D), full),
            pl.BlockSpec((1, D), full),
            pl.BlockSpec((D, D), full),
            pl.BlockSpec((1, D), full),
        ],
        out_specs=[
            pl.BlockSpec((ROWS_BLK, D), row),
            pl.BlockSpec((ROWS_BLK, D), row),
            pl.BlockSpec((NSLICE, ROWS_BLK, DSLICE), lambda i: (0, i, 0)),
        ],
        out_shape=[
            jax.ShapeDtypeStruct((NPAD, D), jnp.bfloat16),
            jax.ShapeDtypeStruct((NPAD, D), jnp.bfloat16),
            jax.ShapeDtypeStruct((NSLICE, NPAD, DSLICE), jnp.bfloat16),
        ],
    )(h, g.reshape(1, D), b.reshape(1, D), Wq, bq.reshape(1, D),
      Wk, bk.reshape(1, D), Wv, bv.reshape(1, D))


@functools.lru_cache(maxsize=None)
def _sc_mesh():
    return plsc.VectorSubcoreMesh(
        core_axis_name="c", subcore_axis_name="s",
        num_cores=SC_CORES, num_subcores=SC_SUB)


def _e1_body(q_hbm, k_hbm, src_hbm, dst_hbm, eatt_hbm, den2_hbm,
             src_big, dst_big, dst2, eatt_big,
             sidx0, sidx1, didx0, didx1,
             qr0, qr1, kr0, kr1,
             ech, zstripe,
             qs0, qs1, ks0, ks1, dsem,
             den_sp):
    sidx = [sidx0, sidx1]
    didx = [didx0, didx1]
    qr = [qr0, qr1]
    kr = [kr0, kr1]
    qs = [qs0, qs1]
    ks = [ks0, ks1]
    c = lax.axis_index("c")
    s = lax.axis_index("s")
    w = s * SC_CORES + c
    base = w * EW1
    for i in range(STRIPE // 16):
        zstripe[pl.ds(i * 16, 16)] = jnp.zeros((16,), jnp.float32)
    pltpu.sync_copy(zstripe, den_sp.at[pl.ds(s * STRIPE, STRIPE)])
    pltpu.sync_copy(src_hbm.at[pl.ds(base, EW1)], src_big)
    pltpu.sync_copy(dst_hbm.at[pl.ds(base, EW1)], dst_big)
    nrow2 = EW1 // 128
    d2descs = [
        pltpu.async_copy(dst_hbm.at[pl.ds(base + j * 128, 128)],
                         dst2.at[j], dsem)
        for j in range(nrow2)
    ]
    for dsc in d2descs:
        dsc.wait()

    def fire(b, ci):
        off = ci * CH1
        for t in range(CH1 // 16):
            sidx[b][pl.ds(t * 16, 16)] = src_big[pl.ds(off + t * 16, 16)]
            didx[b][pl.ds(t * 16, 16)] = dst_big[pl.ds(off + t * 16, 16)]
        pltpu.async_copy(q_hbm.at[didx[b]], qr[b], qs[b])
        pltpu.async_copy(k_hbm.at[sidx[b]], kr[b], ks[b])

    for b in range(RING1):
        fire(b, b)

    def outer(ip, carry):
        for b in range(RING1):
            ci = ip * RING1 + b
            off = ci * CH1
            pltpu.make_async_copy(q_hbm.at[didx[b]], qr[b], qs[b]).wait()
            pltpu.make_async_copy(k_hbm.at[sidx[b]], kr[b], ks[b]).wait()
            for g in range(CH1 // 16):
                ech[pl.ds(g * 16, 16)] = jnp.zeros((16,), jnp.float32)
            for e in range(CH1):
                p = (qr[b][e, pl.ds(0, 32)] * kr[b][e, pl.ds(0, 32)])
                pa, pb = plsc.unpack(p, format=plsc.PackFormat.INTERLEAVED)
                acc = pa + pb
                for j in range(1, D // 32):
                    p = (qr[b][e, pl.ds(j * 32, 32)]
                         * kr[b][e, pl.ds(j * 32, 32)])
                    pa, pb = plsc.unpack(
                        p, format=plsc.PackFormat.INTERLEAVED)
                    acc = acc + pa + pb
                plsc.addupdate_scatter(
                    ech, [jnp.full((16,), e, jnp.int32)], acc)
            for g in range(CH1 // 16):
                eatt_big[pl.ds(off + g * 16, 16)] = jnp.exp(
                    ech[pl.ds(g * 16, 16)])

            @pl.when(ci + RING1 < NCH1)
            def _():
                fire(b, ci + RING1)

        return carry

    lax.fori_loop(0, NCH1 // RING1, outer, 0)
    pltpu.sync_copy(eatt_big, eatt_hbm.at[pl.ds(base, EW1)])
    plsc.subcore_barrier()
    descs = []
    for j in range(nrow2):
        descs.append(pltpu.async_copy(
            eatt_big.at[pl.ds(j * 128, 128)],
            den_sp.at[dst2.at[j]], dsem, add=True))
    for dsc in descs:
        dsc.wait()
    plsc.subcore_barrier()
    pltpu.sync_copy(den_sp.at[pl.ds(s * STRIPE, STRIPE)],
                    den2_hbm.at[pl.ds(c * NPAD + s * STRIPE, STRIPE)])


@functools.lru_cache(maxsize=None)
def _e1():
    return pl.kernel(
        _e1_body,
        out_type=[
            jax.ShapeDtypeStruct((EPAD,), jnp.float32),
            jax.ShapeDtypeStruct((2 * NPAD,), jnp.float32),
        ],
        mesh=_sc_mesh(),
        compiler_params=pltpu.CompilerParams(
            use_tc_tiling_on_sc=False, needs_layout_passes=False),
        scratch_types=(
            [
                pltpu.VMEM((EW1,), jnp.int32),
                pltpu.VMEM((EW1,), jnp.int32),
                pltpu.VMEM((EW1 // 128, 128), jnp.int32),
                pltpu.VMEM((EW1,), jnp.float32),
            ]
            + [pltpu.VMEM((CH1,), jnp.int32)] * 4
            + [pltpu.VMEM((CH1, D), jnp.bfloat16)] * 4
            + [
                pltpu.VMEM((CH1,), jnp.float32),
                pltpu.VMEM((STRIPE,), jnp.float32),
            ]
            + [pltpu.SemaphoreType.DMA] * 5
            + [pltpu.VMEM_SHARED((NPAD,), jnp.float32)]
        ),
    )


def _e2_body(v_hbm, src_hbm, dst_hbm, eatt_hbm, z_hbm, agg_hbm,
             src_big, dst_big,
             ea0, ea1, gv0, gv1, sd0, sd1, vr0, vr1, mr,
             es0, es1, vs0, vs1,
             zrows, agg_sp):
    ea = [ea0, ea1]
    es = [es0, es1]
    gv = [gv0, gv1]
    sd = [sd0, sd1]
    vr = [vr0, vr1]
    vs = [vs0, vs1]
    c = lax.axis_index("c")
    s = lax.axis_index("s")
    ebase = s * EW2
    pltpu.sync_copy(z_hbm, zrows)
    pltpu.sync_copy(src_hbm.at[pl.ds(ebase, EW2)], src_big)
    pltpu.sync_copy(dst_hbm.at[pl.ds(ebase, EW2)], dst_big)

    for rnd in range(NSLICE // SC_CORES):
        r = c * (NSLICE // SC_CORES) + rnd
        rbase = r * NPAD
        for zz in range(STRIPE // 32):
            pltpu.sync_copy(zrows,
                            agg_sp.at[pl.ds(s * STRIPE + zz * 32, 32)])
        plsc.subcore_barrier()

        def fire_v(b, ci):
            off = ci * CH2
            for t in range(CH2 // 16):
                gv[b][pl.ds(t * 16, 16)] = (
                    src_big[pl.ds(off + t * 16, 16)] + rbase)
                sd[b][pl.ds(t * 16, 16)] = dst_big[pl.ds(off + t * 16, 16)]
            pltpu.async_copy(v_hbm.at[gv[b]], vr[b], vs[b])
            pltpu.async_copy(eatt_hbm.at[pl.ds(ebase + off, CH2)],
                             ea[b], es[b])

        for b in range(RING2):
            fire_v(b, b)

        def agg_loop(ip, carry):
            for b in range(RING2):
                ci = ip * RING2 + b
                off = ci * CH2
                pltpu.make_async_copy(v_hbm.at[gv[b]], vr[b],
                                      vs[b]).wait()
                pltpu.make_async_copy(
                    eatt_hbm.at[pl.ds(ebase + off, CH2)], ea[b],
                    es[b]).wait()
                for t in range(CH2 // 16):
                    a16 = ea[b][pl.ds(t * 16, 16)]
                    for i in range(16):
                        e = t * 16 + i
                        av = jnp.full((16,), a16[i], jnp.float32)
                        for j in range(DSLICE // 32):
                            pv = vr[b][e, pl.ds(j * 32, 32)]
                            pa, pb = plsc.unpack(
                                pv, format=plsc.PackFormat.INTERLEAVED)
                            mr[e, pl.ds(j * 32, 16)] = pa * av
                            mr[e, pl.ds(j * 32 + 16, 16)] = pb * av
                pltpu.sync_copy(mr, agg_sp.at[sd[b]], add=True)

                @pl.when(ci + RING2 < NCH2)
                def _():
                    fire_v(b, ci + RING2)

            return carry

        lax.fori_loop(0, NCH2 // RING2, agg_loop, 0)
        plsc.subcore_barrier()
        pltpu.sync_copy(agg_sp.at[pl.ds(s * STRIPE, STRIPE)],
                        agg_hbm.at[pl.ds(rbase + s * STRIPE, STRIPE)])


@functools.lru_cache(maxsize=None)
def _e2():
    return pl.kernel(
        _e2_body,
        out_type=jax.ShapeDtypeStruct((NSLICE * NPAD, DSLICE), jnp.float32),
        mesh=_sc_mesh(),
        compiler_params=pltpu.CompilerParams(
            use_tc_tiling_on_sc=False, needs_layout_passes=False),
        scratch_types=(
            [
                pltpu.VMEM((EW2,), jnp.int32),
                pltpu.VMEM((EW2,), jnp.int32),
            ]
            + [pltpu.VMEM((CH2,), jnp.float32)] * 2
            + [pltpu.VMEM((CH2,), jnp.int32)] * 4
            + [pltpu.VMEM((CH2, DSLICE), jnp.bfloat16)] * 2
            + [pltpu.VMEM((CH2, DSLICE), jnp.float32)]
            + [pltpu.SemaphoreType.DMA] * 4
            + [
                pltpu.VMEM((32, DSLICE), jnp.float32),
                pltpu.VMEM_SHARED((NPAD, DSLICE), jnp.float32),
            ]
        ),
    )


def _post_body(agg_ref, den_ref, h_ref, wo_ref, bo_ref, g_ref, b_ref,
               o_ref):
    acc = jnp.dot(agg_ref[0], wo_ref[pl.ds(0, DSLICE), :],
                  preferred_element_type=jnp.float32)
    for j in range(1, NSLICE):
        acc = acc + jnp.dot(agg_ref[j],
                            wo_ref[pl.ds(j * DSLICE, DSLICE), :],
                            preferred_element_type=jnp.float32)
    den = den_ref[:, 0:1] + den_ref[:, 1:2]
    inv = jnp.where(den > 0.0, 1.0 / den, 1.0)
    a = acc * inv + bo_ref[...]
    o_ref[...] = h_ref[...] + jax.nn.relu(
        _layer_norm_rows(a, g_ref[...], b_ref[...]))


def _post(agg4, den2, h, Wo, bo, g, b):
    row = lambda i: (i, 0)
    full = lambda i: (0, 0)
    return pl.pallas_call(
        _post_body,
        grid=(GRID_ROWS,),
        in_specs=[
            pl.BlockSpec((NSLICE, ROWS_BLK, DSLICE), lambda i: (0, i, 0)),
            pl.BlockSpec((ROWS_BLK, 2), lambda i: (i, 0)),
            pl.BlockSpec((ROWS_BLK, D), row),
            pl.BlockSpec((D, D), full),
            pl.BlockSpec((1, D), full),
            pl.BlockSpec((1, D), full),
            pl.BlockSpec((1, D), full),
        ],
        out_specs=pl.BlockSpec((ROWS_BLK, D), row),
        out_shape=jax.ShapeDtypeStruct((NPAD, D), jnp.float32),
    )(agg4, den2, h, Wo, bo.reshape(1, D), g.reshape(1, D),
      b.reshape(1, D))


def _head_body(h_ref, w1_ref, b1_ref, w2_ref, b2_ref, o_ref):
    z = jax.nn.relu(
        jnp.dot(h_ref[...], w1_ref[...], preferred_element_type=jnp.float32)
        + b1_ref[...]
    )
    o_ref[...] = (
        jnp.dot(z, w2_ref[...], preferred_element_type=jnp.float32)
        + b2_ref[...]
    )


def _head(h, W1, b1, W2, b2):
    row = lambda i: (i, 0)
    full = lambda i: (0, 0)
    return pl.pallas_call(
        _head_body,
        grid=(GRID_ROWS,),
        in_specs=[
            pl.BlockSpec((ROWS_BLK, D), row),
            pl.BlockSpec((D, 2 * D), full),
            pl.BlockSpec((1, 2 * D), full),
            pl.BlockSpec((2 * D, C), full),
            pl.BlockSpec((1, C), full),
        ],
        out_specs=pl.BlockSpec((ROWS_BLK, C), row),
        out_shape=jax.ShapeDtypeStruct((NPAD, C), jnp.float32),
    )(h, W1, b1.reshape(1, 2 * D), W2, b2.reshape(1, C))


_PERM = []
for _cb in range(D // 32):
    _PERM += [_cb * 32 + 2 * _t for _t in range(16)]
    _PERM += [_cb * 32 + 2 * _t + 1 for _t in range(16)]


def kernel(x, edge_index, W_emb, b_emb, ln1_g, ln1_b, ln2_g, ln2_b,
           Wq, bq, Wk, bk, Wv, bv, Wo, bo, W1, b1, W2, b2):
    pad = jnp.full((EPAD - E,), N, jnp.int32)
    src = jnp.concatenate([edge_index[0], pad])
    dst = jnp.concatenate([edge_index[1], pad])
    zrows = jnp.zeros((32, DSLICE), jnp.float32)
    perm = jnp.array(_PERM, jnp.int32)
    x_pad = jnp.pad(x, ((0, NPAD - N), (0, 0)))
    h = _embed(x_pad, W_emb, b_emb)
    for i in range(L):
        q, k, v4 = _qkv(h, ln1_g[i], ln1_b[i], Wq[i], bq[i],
                        Wk[i], bk[i], Wv[i], bv[i])
        eatt, den2 = _e1()(q, k, src, dst)
        agg = _e2()(v4.reshape(NSLICE * NPAD, DSLICE), src, dst, eatt,
                    zrows)
        den2t = den2.reshape(2, NPAD).T
        h = _post(agg.reshape(NSLICE, NPAD, DSLICE), den2t, h,
                  jnp.take(Wo[i], perm, axis=0), bo[i],
                  ln2_g[i], ln2_b[i])
    logits = _head(h, W1, b1, W2, b2)
    return logits[:N]
